# speedup vs baseline: 7.1849x; 7.1849x over previous
import functools

import jax
import jax.numpy as jnp
from jax import lax
from jax.experimental import pallas as pl
from jax.experimental.pallas import tpu as pltpu
from jax.experimental.pallas import tpu_sc as plsc

_NC = 2
_NS = 16
_K = 80


def _sc_segment_sum(x, e4, zeros_pad):
    n, d = x.shape
    nblk = e4.shape[2]
    n_pad = zeros_pad.shape[0]
    rows_per_sub = n_pad // _NS

    mesh = plsc.VectorSubcoreMesh(core_axis_name="c", subcore_axis_name="s")

    @functools.partial(
        pl.kernel,
        out_type=jax.ShapeDtypeStruct((_NC, n_pad, d), jnp.float32),
        mesh=mesh,
        scratch_types=[
            pltpu.VMEM_SHARED((n_pad, d), jnp.float32),
            pltpu.VMEM((nblk, _K), jnp.int32),
            pltpu.VMEM((nblk, _K), jnp.int32),
            pltpu.VMEM((_K, d), jnp.float32),
        ],
    )
    def k(x_hbm, e_hbm, z_hbm, o_hbm, acc, sidx, didx, rows):
        c = lax.axis_index("c")
        s = lax.axis_index("s")
        wid = c * _NS + s

        r0 = s * rows_per_sub
        pltpu.sync_copy(z_hbm.at[pl.ds(r0, rows_per_sub)],
                        acc.at[pl.ds(r0, rows_per_sub)])
        pltpu.sync_copy(e_hbm.at[0, wid], sidx)
        pltpu.sync_copy(e_hbm.at[1, wid], didx)
        plsc.subcore_barrier()

        @pl.loop(0, nblk)
        def _(b):
            pltpu.sync_copy(x_hbm.at[sidx.at[b]], rows)
            pltpu.sync_copy(rows, acc.at[didx.at[b]], add=True)

        plsc.subcore_barrier()
        pltpu.sync_copy(acc.at[pl.ds(r0, rows_per_sub)],
                        o_hbm.at[c, pl.ds(r0, rows_per_sub)])

    return k(x, e4, zeros_pad)


def _tc_linear(x, agg, w, b, relu):
    n, d = x.shape
    h = w.shape[1]
    br = 1000

    def kern(x_ref, a_ref, w_ref, b_ref, o_ref):
        t = x_ref[...] + a_ref[0] + a_ref[1]
        y = jnp.dot(t, w_ref[...], preferred_element_type=jnp.float32)
        y = y + b_ref[...]
        o_ref[...] = jnp.maximum(y, 0.0) if relu else y

    return pl.pallas_call(
        kern,
        grid=(n // br,),
        in_specs=[
            pl.BlockSpec((br, d), lambda i: (i, 0)),
            pl.BlockSpec((_NC, br, d), lambda i: (0, i, 0)),
            pl.BlockSpec((d, h), lambda i: (0, 0)),
            pl.BlockSpec((1, h), lambda i: (0, 0)),
        ],
        out_specs=pl.BlockSpec((br, h), lambda i: (i, 0)),
        out_shape=jax.ShapeDtypeStruct((n, h), jnp.float32),
    )(x, agg, w, b.reshape(1, h))


def kernel(features, edge_index, W1, b1, W2, b2):
    n, d = features.shape
    e = edge_index.shape[1]
    c = W2.shape[1]
    n_workers = _NC * _NS
    n_per = e // n_workers
    nblk = n_per // _K
    n_pad = -(-n // 128) * 128

    e4 = edge_index.reshape(2, n_workers, nblk, _K)
    zeros_pad = jnp.zeros((n_pad, d), jnp.float32)

    agg1 = _sc_segment_sum(features, e4, zeros_pad)
    x = _tc_linear(features, agg1[:, :n], W1, b1, relu=True)
    agg2 = _sc_segment_sum(x, e4, zeros_pad)

    c_pad = -(-c // 128) * 128
    w2p = jnp.pad(W2, ((0, 0), (0, c_pad - c)))
    b2p = jnp.pad(b2, (0, c_pad - c))
    logits = _tc_linear(x, agg2[:, :n], w2p, b2p, relu=False)
    return logits[:, :c]
